# baseline (speedup 1.0000x reference)
import functools
import math

import jax
import jax.numpy as jnp
from jax import lax
from jax.experimental import pallas as pl
from jax.experimental.pallas import tpu as pltpu
from jax.experimental.pallas import tpu_sc as plsc

NPRED = 1_000_000
BATCH = 16384
DUR_N = 11
TOTAL = float(DUR_N - 1)

R, C = 2000, 1000
NCHUNK = 10
RBLK = R // NCHUNK

_NC, _NS = 2, 16
_NW = _NC * _NS
_BPW = BATCH // _NW
_IDX_ROWS = _BPW // 128

_EPS = float(jnp.finfo(jnp.float32).eps)
_LOGC = [
    math.lgamma(DUR_N) - math.lgamma(j + 1.0) - math.lgamma(TOTAL - j + 1.0)
    for j in range(DUR_N)
]


def _min_body(x_ref, o_ref, acc_ref):
    i = pl.program_id(0)
    x = x_ref[...]
    lane = lax.broadcasted_iota(jnp.int32, x.shape, 1)
    sq = x * x
    m = jnp.min(jnp.where(lane % 2 == 0, sq, jnp.inf))

    @pl.when(i == 0)
    def _():
        acc_ref[0, 0] = m

    @pl.when(i > 0)
    def _():
        acc_ref[0, 0] = jnp.minimum(acc_ref[0, 0], m)

    @pl.when(i == NCHUNK - 1)
    def _():
        o_ref[0, 0] = acc_ref[0, 0]


def _anch_body(m_ref, x_ref, o_ref):
    x = x_ref[...]
    sq = x * x
    lane = lax.broadcasted_iota(jnp.int32, x.shape, 1)
    o_ref[...] = jnp.where(lane % 2 == 0, sq - m_ref[0, 0], sq)


def _gather_sc(pred_flat, idx1, idx2):
    mesh = plsc.VectorSubcoreMesh(core_axis_name="c", subcore_axis_name="s")

    @functools.partial(
        pl.kernel,
        mesh=mesh,
        out_type=jax.ShapeDtypeStruct((4, BATCH), jnp.float32),
        scratch_types=[
            pltpu.VMEM((_BPW,), jnp.int32),
            pltpu.VMEM((_IDX_ROWS, 128), jnp.int32),
            pltpu.VMEM((_BPW,), jnp.float32),
            pltpu.SemaphoreType.DMA,
        ],
    )
    def k(tab_hbm, i1_hbm, i2_hbm, out_hbm, raw_v, sidx_v, rows_v, sem):
        wid = lax.axis_index("s") * _NC + lax.axis_index("c")
        base = wid * _BPW
        for t, src in enumerate((i1_hbm, i2_hbm)):
            pltpu.sync_copy(src.at[pl.ds(base, _BPW)], raw_v)
            for parity in range(2):
                for i in range(_BPW // 16):
                    v = raw_v[pl.ds(i * 16, 16)]
                    sidx_v[i // 8, pl.ds((i % 8) * 16, 16)] = v * 2 + parity
                descs = [
                    pltpu.async_copy(
                        tab_hbm.at[sidx_v.at[j]],
                        rows_v.at[pl.ds(j * 128, 128)],
                        sem,
                    )
                    for j in range(_IDX_ROWS)
                ]
                for d in descs:
                    d.wait()
                pltpu.sync_copy(
                    rows_v, out_hbm.at[2 * t + parity, pl.ds(base, _BPW)]
                )

    return k(pred_flat, idx1, idx2)


def _small_body(m_ref, k_ref, g_ref,
                b1_ref, d1_ref, b2_ref, d2_ref, p1_ref, p2_ref):
    minv = m_ref[0, 0]
    kk = k_ref[0, 0]
    for t, (b_ref, d_ref, p_ref) in enumerate(
            ((b1_ref, d1_ref, p1_ref), (b2_ref, d2_ref, p2_ref))):
        a = g_ref[2 * t]
        d = g_ref[2 * t + 1]
        dur = d * d
        b_ref[...] = a * a - minv
        d_ref[...] = dur
        x = kk * jnp.log(dur)
        p = jax.nn.sigmoid(x)
        p = jnp.clip(p, _EPS, 1.0 - _EPS)
        logits = jnp.log(p) - jnp.log1p(-p)
        neg_max = jnp.minimum(logits, 0.0)
        base = TOTAL * neg_max - TOTAL * jnp.log(
            jnp.exp(neg_max) + jnp.exp(-logits + neg_max))
        for j in range(DUR_N):
            p_ref[:, :, j] = _LOGC[j] + float(j) * logits + base


def kernel(idx1, idx2, pred_tensor, k):
    flat = pred_tensor.reshape(R, C)
    pred1d = pred_tensor.reshape(NPRED * 2)

    minv = pl.pallas_call(
        _min_body,
        grid=(NCHUNK,),
        in_specs=[pl.BlockSpec((RBLK, C), lambda i: (i, 0))],
        out_specs=pl.BlockSpec(memory_space=pltpu.SMEM),
        out_shape=jax.ShapeDtypeStruct((1, 1), jnp.float32),
        scratch_shapes=[pltpu.SMEM((1, 1), jnp.float32)],
    )(flat)

    g = _gather_sc(pred1d, idx1, idx2)

    anch_flat = pl.pallas_call(
        _anch_body,
        grid=(NCHUNK,),
        in_specs=[
            pl.BlockSpec(memory_space=pltpu.SMEM),
            pl.BlockSpec((RBLK, C), lambda i: (i, 0)),
        ],
        out_specs=pl.BlockSpec((RBLK, C), lambda i: (i, 0)),
        out_shape=jax.ShapeDtypeStruct((R, C), jnp.float32),
    )(minv, flat)

    g3 = g.reshape(4, 128, 128)
    k2 = k.reshape(1, 1)
    grid_s = 16
    sub = 128 // grid_s
    b1, d1, b2, d2, p1, p2 = pl.pallas_call(
        _small_body,
        grid=(grid_s,),
        in_specs=[
            pl.BlockSpec(memory_space=pltpu.SMEM),
            pl.BlockSpec(memory_space=pltpu.SMEM),
            pl.BlockSpec((4, sub, 128), lambda i: (0, i, 0)),
        ],
        out_specs=[
            pl.BlockSpec((sub, 128), lambda i: (i, 0)),
            pl.BlockSpec((sub, 128), lambda i: (i, 0)),
            pl.BlockSpec((sub, 128), lambda i: (i, 0)),
            pl.BlockSpec((sub, 128), lambda i: (i, 0)),
            pl.BlockSpec((sub, 128, DUR_N), lambda i: (i, 0, 0)),
            pl.BlockSpec((sub, 128, DUR_N), lambda i: (i, 0, 0)),
        ],
        out_shape=[
            jax.ShapeDtypeStruct((128, 128), jnp.float32),
            jax.ShapeDtypeStruct((128, 128), jnp.float32),
            jax.ShapeDtypeStruct((128, 128), jnp.float32),
            jax.ShapeDtypeStruct((128, 128), jnp.float32),
            jax.ShapeDtypeStruct((128, 128, DUR_N), jnp.float32),
            jax.ShapeDtypeStruct((128, 128, DUR_N), jnp.float32),
        ],
    )(minv, k2, g3)

    return (
        b1.reshape(BATCH),
        d1.reshape(BATCH),
        b2.reshape(BATCH),
        d2.reshape(BATCH),
        p1.reshape(BATCH, DUR_N),
        p2.reshape(BATCH, DUR_N),
        anch_flat.reshape(NPRED, 2),
    )
